# Initial kernel scaffold; baseline (speedup 1.0000x reference)
import jax
import jax.numpy as jnp
from jax.experimental import pallas as pl


def kernel(positions, x, W1, b1, W2, b2, Wf, bf, Wp, bp):
    raise NotImplementedError("write your pallas kernel here")


# speedup vs baseline: 9.1802x; 9.1802x over previous
import functools

import numpy as np
import jax
import jax.numpy as jnp
from jax import lax
from jax.experimental import pallas as pl
from jax.experimental.pallas import tpu as pltpu
from jax.experimental.pallas import tpu_sc as plsc

RES = 32
CH = 4
PS = 3
PD = 108
B = 4
S = 2048
DIN = 64

NPTS = B * S
NOFF = PS ** 3
NOFFP = 32
ROWSP = NPTS * NOFFP
GRID = B * RES ** 3
RW = 8
YW = NOFFP * RW

NC, NS = 2, 16
NW = NC * NS
RPW = ROWSP // NW
ICH = 128
NIC = RPW // ICH
RB = 4096
NB = RPW // RB
CPB = RB // ICH
ZR = GRID // NS

MLP_BLK = 512
PROJ_BLK = 2048

_MP = np.zeros((PD, YW), np.float32)
for _j in range(PD):
    _MP[_j, RW * (_j // 4) + (_j // 27)] = 1.0
_CNT = np.zeros((YW,), np.float32)
for _j in range(NOFF):
    _CNT[RW * (_j // 4) + 4] += 1.0

def _gelu(v):
    return 0.5 * v * (1.0 + lax.erf(v * np.float32(0.7071067811865476)))


def _mlp_body(pos_ref, x_ref, w1_ref, b1_ref, w2_ref, b2_ref, wf_ref, bf_ref,
              y_ref, v_ref):
    h = _gelu(jnp.dot(x_ref[...], w1_ref[...],
                      preferred_element_type=jnp.float32) + b1_ref[...])
    h = _gelu(jnp.dot(h, w2_ref[...],
                      preferred_element_type=jnp.float32) + b2_ref[...])
    y_ref[...] = jnp.dot(h, wf_ref[...],
                         preferred_element_type=jnp.float32) + bf_ref[...]

    p = (pos_ref[...] * np.float32(RES)).astype(jnp.int32)
    row0 = pl.program_id(0) * MLP_BLK
    bidx = (row0 + lax.broadcasted_iota(jnp.int32, (MLP_BLK, 1), 0)) // S
    kk = lax.broadcasted_iota(jnp.int32, (1, NOFFP), 1)
    d0 = (kk // 3) % 3 - 1
    d1 = kk // 9 - 1
    d2 = kk % 3 - 1
    c0 = jnp.clip(p[:, 0:1] + d0, 0, RES - 1)
    c1 = jnp.clip(p[:, 1:2] + d1, 0, RES - 1)
    c2 = jnp.clip(p[:, 2:3] + d2, 0, RES - 1)
    vid = ((bidx * RES + c0) * RES + c1) * RES + c2
    v_ref[...] = jnp.where(kk < NOFF, vid, 0)


_mlp_call = pl.pallas_call(
    _mlp_body,
    grid=(NPTS // MLP_BLK,),
    in_specs=[
        pl.BlockSpec((MLP_BLK, 3), lambda i: (i, 0)),
        pl.BlockSpec((MLP_BLK, DIN), lambda i: (i, 0)),
        pl.BlockSpec((DIN, PD * 8), lambda i: (0, 0)),
        pl.BlockSpec((1, PD * 8), lambda i: (0, 0)),
        pl.BlockSpec((PD * 8, PD * 4), lambda i: (0, 0)),
        pl.BlockSpec((1, PD * 4), lambda i: (0, 0)),
        pl.BlockSpec((PD * 4, YW), lambda i: (0, 0)),
        pl.BlockSpec((1, YW), lambda i: (0, 0)),
    ],
    out_specs=[
        pl.BlockSpec((MLP_BLK, YW), lambda i: (i, 0)),
        pl.BlockSpec((MLP_BLK, NOFFP), lambda i: (i, 0)),
    ],
    out_shape=[
        jax.ShapeDtypeStruct((NPTS, YW), jnp.float32),
        jax.ShapeDtypeStruct((NPTS, NOFFP), jnp.int32),
    ],
    compiler_params=pltpu.CompilerParams(
        dimension_semantics=("parallel",)),
)


def _scatter_body(v_hbm, u_hbm, z_hbm, out_hbm, idx_v, rows_v, acc):
    cid = lax.axis_index("c")
    sid = lax.axis_index("s")
    wid = sid * NC + cid
    pltpu.sync_copy(v_hbm.at[pl.ds(wid * NIC, NIC)], idx_v)
    pltpu.sync_copy(z_hbm, acc.at[pl.ds(sid * ZR, ZR)])
    plsc.subcore_barrier()

    for bi in range(NB):
        pltpu.sync_copy(u_hbm.at[pl.ds(wid * RPW + bi * RB, RB)], rows_v)

        def body(j, carry, bi=bi):
            pltpu.sync_copy(rows_v.at[pl.ds(j * ICH, ICH)],
                            acc.at[idx_v.at[bi * CPB + j]], add=True)
            return carry

        lax.fori_loop(0, CPB, body, 0)
    plsc.subcore_barrier()
    pltpu.sync_copy(acc.at[pl.ds(sid * ZR, ZR)],
                    out_hbm.at[cid].at[pl.ds(sid * ZR, ZR)])


@functools.cache
def _get_scatter_kernel():
    mesh = plsc.VectorSubcoreMesh(
        core_axis_name="c", subcore_axis_name="s",
        num_cores=NC, num_subcores=NS)
    return pl.kernel(
        _scatter_body,
        out_type=jax.ShapeDtypeStruct((NC, GRID, RW), jnp.float32),
        mesh=mesh,
        scratch_types=[
            pltpu.VMEM((NIC, ICH), jnp.int32),
            pltpu.VMEM((RB, RW), jnp.float32),
            pltpu.VMEM_SHARED((GRID, RW), jnp.float32),
        ],
        compiler_params=pltpu.CompilerParams(use_tc_tiling_on_sc=False),
    )


def _proj_body(acc_ref, wp_ref, bp_ref, o_ref):
    a = acc_ref[0] + acc_ref[1]
    o_ref[...] = jnp.dot(a, wp_ref[...],
                         preferred_element_type=jnp.float32) + bp_ref[...]


_proj_call = pl.pallas_call(
    _proj_body,
    grid=(GRID // PROJ_BLK,),
    in_specs=[
        pl.BlockSpec((NC, PROJ_BLK, RW), lambda i: (0, i, 0)),
        pl.BlockSpec((RW, PD), lambda i: (0, 0)),
        pl.BlockSpec((1, PD), lambda i: (0, 0)),
    ],
    out_specs=pl.BlockSpec((PROJ_BLK, PD), lambda i: (i, 0)),
    out_shape=jax.ShapeDtypeStruct((GRID, PD), jnp.float32),
    compiler_params=pltpu.CompilerParams(
        dimension_semantics=("parallel",)),
)


def kernel(positions, x, W1, b1, W2, b2, Wf, bf, Wp, bp):
    wf216 = Wf @ jnp.asarray(_MP)
    bf216 = (bf @ jnp.asarray(_MP) + jnp.asarray(_CNT)).reshape(1, -1)
    y, v = _mlp_call(positions.reshape(NPTS, 3), x.reshape(NPTS, DIN),
                     W1, b1.reshape(1, -1), W2, b2.reshape(1, -1),
                     wf216, bf216)
    u = y.reshape(ROWSP, RW)
    vr = v.reshape(NW * NIC, ICH)
    zeros = jnp.zeros((ZR, RW), jnp.float32)
    accs = _get_scatter_kernel()(vr, u, zeros)
    wp8 = jnp.concatenate([Wp, jnp.zeros((3, PD), Wp.dtype)], axis=0)
    out = _proj_call(accs, wp8, bp.reshape(1, -1))
    return out.reshape(B, RES, RES, RES, PD)
